# Initial kernel scaffold; baseline (speedup 1.0000x reference)
import jax
import jax.numpy as jnp
from jax.experimental import pallas as pl


def kernel(x, edge_index, dropout, W0, a0, W1, a1, W2, a2, W3, a3, W_out, a_out):
    raise NotImplementedError("write your pallas kernel here")


# speedup vs baseline: 15.7459x; 15.7459x over previous
import functools

import jax
import jax.numpy as jnp
from jax import lax
from jax.experimental import pallas as pl
from jax.experimental.pallas import tpu as pltpu
from jax.experimental.pallas import tpu_sc as plsc

N = 10000
NP = 10240
E = 160000
F = 256
NH = 4
L = 16
NC = 2
NS = 16
ET = E // NS
K = 80
NCHUNK = ET // K
RT = NP // NS
RC = 40
NRC = RT // RC
FH = F // NC
NV = FH // L


SC_E = 2000
NSC = ET // SC_E
NCK = SC_E // K


def _sc_gal_body(nh, table, src_hbm, dst_hbm, s1_hbm, s2_hbm, out_hbm,
                 src1d, dst1d, gix, dix, s1g, s2g, ex1d, rowbuf, cbuf, denbuf,
                 s1_sh, s2_sh, acc_sh, den_sh, gsem, ssem, dsem, xsem):
    c = lax.axis_index("c")
    s = lax.axis_index("s")
    rbase = pl.multiple_of(s * RT, 8)

    zeros16 = jnp.zeros((L,), jnp.float32)

    def zero_cbuf(i, _):
        for j in range(NV):
            cbuf[i, pl.ds(j * L, L)] = zeros16
        return 0

    def zero_den(i, _):
        denbuf[pl.ds(i * L, L)] = zeros16
        return 0

    lax.fori_loop(0, RC, zero_cbuf, 0)
    lax.fori_loop(0, RT // L, zero_den, 0)

    def head(h, _):
        for r in range(NRC):
            pltpu.sync_copy(cbuf, acc_sh.at[pl.ds(rbase + r * RC, RC)])
        pltpu.sync_copy(denbuf, den_sh.at[pl.ds(rbase, RT)])

        sbase = pl.multiple_of(h * NP, 8)

        @pl.when(s == 0)
        def _load_scores():
            pltpu.sync_copy(s1_hbm.at[pl.ds(sbase, NP)], s1_sh)
            pltpu.sync_copy(s2_hbm.at[pl.ds(sbase, NP)], s2_sh)

        plsc.subcore_barrier()

        stride = jnp.int32(2 * nh)
        off = jnp.int32(2) * h + c

        def superchunk(u, _):
            eb = pl.multiple_of(s * ET + u * SC_E, 8)
            pltpu.sync_copy(src_hbm.at[pl.ds(eb, SC_E)], src1d)
            pltpu.sync_copy(dst_hbm.at[pl.ds(eb, SC_E)], dst1d)
            pltpu.async_copy(s1_sh.at[src1d], s1g, xsem).wait()
            pltpu.async_copy(s2_sh.at[dst1d], s2g, xsem).wait()

            def exloop(i, _):
                uu = s1g[pl.ds(i * L, L)] + s2g[pl.ds(i * L, L)]
                ee = jnp.where(uu >= 0.0, uu, uu * jnp.float32(0.2))
                ex1d[pl.ds(i * L, L)] = jnp.exp(ee)
                return 0
            lax.fori_loop(0, SC_E // L, exloop, 0)

            def gixloop(i, _):
                r_ = i // (K // L)
                o_ = (i % (K // L)) * L
                sv = src1d[pl.ds(i * L, L)]
                dv = dst1d[pl.ds(i * L, L)]
                gix[r_, pl.ds(o_, L)] = sv * stride + off
                dix[r_, pl.ds(o_, L)] = dv
                return 0
            lax.fori_loop(0, SC_E // L, gixloop, 0)

            def start_gather(r):
                return pltpu.async_copy(
                    table.at[gix.at[r]], rowbuf.at[r % 2], gsem)

            def scale_chunk(r):
                def scale(e_, _):
                    exs = plsc.load_gather(
                        ex1d, [jnp.full((L,), r * K + e_, jnp.int32)])
                    for j in range(NV):
                        rowbuf[r % 2, e_, pl.ds(j * L, L)] = (
                            rowbuf[r % 2, e_, pl.ds(j * L, L)] * exs)
                    return 0
                lax.fori_loop(0, K, scale, 0)

            pend_s = [None] * NCK
            pend_d = [None] * NCK
            g_cur = start_gather(0)
            for r in range(NCK):
                if r + 1 < NCK:
                    if r >= 1:
                        pend_s[r - 1].wait()
                        pend_d[r - 1].wait()
                    g_next = start_gather(r + 1)
                g_cur.wait()
                scale_chunk(r)
                pend_s[r] = pltpu.async_copy(
                    rowbuf.at[r % 2], acc_sh.at[dix.at[r]], ssem, add=True)
                pend_d[r] = pltpu.async_copy(
                    ex1d.at[pl.ds(r * K, K)], den_sh.at[dix.at[r]], dsem,
                    add=True)
                if r + 1 < NCK:
                    g_cur = g_next
            pend_s[NCK - 2].wait()
            pend_d[NCK - 2].wait()
            pend_s[NCK - 1].wait()
            pend_d[NCK - 1].wait()
            return 0

        lax.fori_loop(0, NSC, superchunk, 0)
        plsc.subcore_barrier()

        pltpu.sync_copy(den_sh.at[pl.ds(rbase, RT)], denbuf)
        col0 = pl.multiple_of(h * jnp.int32(F) + c * FH, FH)
        for r in range(NRC):
            pltpu.sync_copy(acc_sh.at[pl.ds(rbase + r * RC, RC)], cbuf)

            def scale_out(i, _):
                den = plsc.load_gather(
                    denbuf, [jnp.full((L,), r * RC + i, jnp.int32)])
                rec = jnp.float32(1.0) / (den + jnp.float32(1e-16))
                for j in range(NV):
                    cbuf[i, pl.ds(j * L, L)] = cbuf[i, pl.ds(j * L, L)] * rec
                return 0
            lax.fori_loop(0, RC, scale_out, 0)
            pltpu.sync_copy(
                cbuf,
                out_hbm.at[pl.ds(rbase + r * RC, RC), pl.ds(col0, FH)])
        lax.fori_loop(0, RC, zero_cbuf, 0)
        lax.fori_loop(0, RT // L, zero_den, 0)
        return 0

    lax.fori_loop(0, nh, head, 0)


def _make_sc_gal(nh):
    mesh = plsc.VectorSubcoreMesh(core_axis_name="c", subcore_axis_name="s",
                                  num_cores=NC, num_subcores=NS)
    scratch = [
        pltpu.VMEM((SC_E,), jnp.int32),
        pltpu.VMEM((SC_E,), jnp.int32),
        pltpu.VMEM((NCK, K), jnp.int32),
        pltpu.VMEM((NCK, K), jnp.int32),
        pltpu.VMEM((SC_E,), jnp.float32),
        pltpu.VMEM((SC_E,), jnp.float32),
        pltpu.VMEM((SC_E,), jnp.float32),
        pltpu.VMEM((2, K, FH), jnp.float32),
        pltpu.VMEM((RC, FH), jnp.float32),
        pltpu.VMEM((RT,), jnp.float32),
        pltpu.VMEM_SHARED((NP,), jnp.float32),
        pltpu.VMEM_SHARED((NP,), jnp.float32),
        pltpu.VMEM_SHARED((NP, FH), jnp.float32),
        pltpu.VMEM_SHARED((NP,), jnp.float32),
        pltpu.SemaphoreType.DMA,
        pltpu.SemaphoreType.DMA,
        pltpu.SemaphoreType.DMA,
        pltpu.SemaphoreType.DMA,
    ]
    body = functools.partial(_sc_gal_body, nh)

    def run(table, src, dst, s1, s2):
        return pl.kernel(
            body,
            out_type=jax.ShapeDtypeStruct((NP, F * nh), jnp.float32),
            mesh=mesh,
            compiler_params=pltpu.CompilerParams(needs_layout_passes=False),
            scratch_types=scratch,
        )(table, src, dst, s1, s2)

    return run


def _tc_scorevec_body(w0, w1, w2, w3, a0, a1, a2, a3, wo, ao, v8, vo):
    ws = (w0, w1, w2, w3)
    as_ = (a0, a1, a2, a3)
    for h in range(NH):
        v8[:, 2 * h:2 * h + 2] = jnp.dot(
            ws[h][...], as_[h][...], preferred_element_type=jnp.float32)
    vo[...] = jnp.dot(wo[...], ao[...], preferred_element_type=jnp.float32)


def _tc_feat_body(x_ref, wcat_ref, v8_ref, h_ref, s_ref):
    xb = x_ref[...]
    h_ref[...] = jnp.dot(xb, wcat_ref[...], preferred_element_type=jnp.float32)
    s_ref[...] = jnp.dot(xb, v8_ref[...], preferred_element_type=jnp.float32)


def _tc_out_body(hcat_ref, wout_ref, vo_ref, h_ref, s_ref):
    hb = hcat_ref[...]
    h_ref[...] = jnp.dot(hb, wout_ref[...], preferred_element_type=jnp.float32)
    s_ref[...] = jnp.dot(hb, vo_ref[...], preferred_element_type=jnp.float32)


def _tc_act_body(g_ref, o_ref):
    g = g_ref[...]
    g = jnp.where(g > 0.0, g, jnp.exp(g) - 1.0)
    m = jnp.max(g, axis=1, keepdims=True)
    ex = jnp.exp(g - m)
    o_ref[...] = ex / jnp.sum(ex, axis=1, keepdims=True)


def kernel(x, edge_index, dropout, W0, a0, W1, a1, W2, a2, W3, a3,
           W_out, a_out):
    f32 = jnp.float32
    src = edge_index[0]
    dst = edge_index[1]

    A2 = [jnp.stack([a[:F], a[F:]], axis=1) for a in (a0, a1, a2, a3)]
    Ao2 = jnp.stack([a_out[:F], a_out[F:]], axis=1)
    Wcat = jnp.concatenate([W0, W1, W2, W3], axis=1)

    v8, vo = pl.pallas_call(
        _tc_scorevec_body,
        out_shape=(jax.ShapeDtypeStruct((F, 2 * NH), f32),
                   jax.ShapeDtypeStruct((NH * F, 2), f32)),
    )(W0, W1, W2, W3, A2[0], A2[1], A2[2], A2[3], W_out, Ao2)

    xp = jnp.pad(x, ((0, NP - N), (0, 0)))
    R = 1024
    grid = (NP // R,)
    H, S = pl.pallas_call(
        _tc_feat_body,
        grid=grid,
        in_specs=[
            pl.BlockSpec((R, F), lambda i: (i, 0)),
            pl.BlockSpec((F, NH * F), lambda i: (0, 0)),
            pl.BlockSpec((F, 2 * NH), lambda i: (0, 0)),
        ],
        out_specs=[
            pl.BlockSpec((R, NH * F), lambda i: (i, 0)),
            pl.BlockSpec((R, 2 * NH), lambda i: (i, 0)),
        ],
        out_shape=(jax.ShapeDtypeStruct((NP, NH * F), f32),
                   jax.ShapeDtypeStruct((NP, 2 * NH), f32)),
    )(xp, Wcat, v8)

    table4 = H.reshape(NP * 2 * NH, FH)
    s1h = jnp.transpose(S[:, 0::2]).reshape(-1)
    s2h = jnp.transpose(S[:, 1::2]).reshape(-1)
    hcat = _sc_gal4(table4, src, dst, s1h, s2h)

    ho, So = pl.pallas_call(
        _tc_out_body,
        grid=grid,
        in_specs=[
            pl.BlockSpec((R, NH * F), lambda i: (i, 0)),
            pl.BlockSpec((NH * F, F), lambda i: (0, 0)),
            pl.BlockSpec((NH * F, 2), lambda i: (0, 0)),
        ],
        out_specs=[
            pl.BlockSpec((R, F), lambda i: (i, 0)),
            pl.BlockSpec((R, 2), lambda i: (i, 0)),
        ],
        out_shape=(jax.ShapeDtypeStruct((NP, F), f32),
                   jax.ShapeDtypeStruct((NP, 2), f32)),
    )(hcat, W_out, vo)

    table1 = ho.reshape(NP * 2, FH)
    s1o = So[:, 0]
    s2o = So[:, 1]
    g = _sc_gal1(table1, src, dst, s1o, s2o)

    out = pl.pallas_call(
        _tc_act_body,
        grid=grid,
        in_specs=[pl.BlockSpec((R, F), lambda i: (i, 0))],
        out_specs=pl.BlockSpec((R, F), lambda i: (i, 0)),
        out_shape=jax.ShapeDtypeStruct((NP, F), f32),
    )(g)
    return out[:N]


_sc_gal4 = _make_sc_gal(NH)
_sc_gal1 = _make_sc_gal(1)


# speedup vs baseline: 15.8234x; 1.0049x over previous
import functools

import jax
import jax.numpy as jnp
from jax import lax
from jax.experimental import pallas as pl
from jax.experimental.pallas import tpu as pltpu
from jax.experimental.pallas import tpu_sc as plsc

N = 10000
NP = 10240
E = 160000
F = 256
NH = 4
L = 16
NC = 2
NS = 16
ET = E // NS
K = 80
NCHUNK = ET // K
RT = NP // NS
RC = 40
NRC = RT // RC
FH = F // NC
NV = FH // L


SC_E = 2000
NSC = ET // SC_E
NCK = SC_E // K


def _sc_gal_body(nh, table, src_hbm, dst_hbm, s1_hbm, s2_hbm, out_hbm,
                 src1d, dst1d, gix, dix, s1g, s2g, ex1d, rowbuf, cbuf, denbuf,
                 s1_sh, s2_sh, acc_sh, den_sh, gsem, ssem, dsem, xsem):
    c = lax.axis_index("c")
    s = lax.axis_index("s")
    rbase = pl.multiple_of(s * RT, 8)

    zeros16 = jnp.zeros((L,), jnp.float32)

    def zero_cbuf(i, _):
        for j in range(NV):
            cbuf[i, pl.ds(j * L, L)] = zeros16
        return 0

    def zero_den(i, _):
        denbuf[pl.ds(i * L, L)] = zeros16
        return 0

    lax.fori_loop(0, RC, zero_cbuf, 0)
    lax.fori_loop(0, RT // L, zero_den, 0)

    def head(h, _):
        for r in range(NRC):
            pltpu.sync_copy(cbuf, acc_sh.at[pl.ds(rbase + r * RC, RC)])
        pltpu.sync_copy(denbuf, den_sh.at[pl.ds(rbase, RT)])

        sbase = pl.multiple_of(h * NP, 8)

        @pl.when(s == 0)
        def _load_scores():
            pltpu.sync_copy(s1_hbm.at[pl.ds(sbase, NP)], s1_sh)
            pltpu.sync_copy(s2_hbm.at[pl.ds(sbase, NP)], s2_sh)

        plsc.subcore_barrier()

        stride = jnp.int32(2 * nh)
        off = jnp.int32(2) * h + c

        def superchunk(u, _):
            eb = pl.multiple_of(s * ET + u * SC_E, 8)
            pltpu.sync_copy(src_hbm.at[pl.ds(eb, SC_E)], src1d)
            pltpu.sync_copy(dst_hbm.at[pl.ds(eb, SC_E)], dst1d)
            pltpu.async_copy(s1_sh.at[src1d], s1g, xsem).wait()
            pltpu.async_copy(s2_sh.at[dst1d], s2g, xsem).wait()

            def exloop(i, _):
                uu = s1g[pl.ds(i * L, L)] + s2g[pl.ds(i * L, L)]
                ee = jnp.where(uu >= 0.0, uu, uu * jnp.float32(0.2))
                ex1d[pl.ds(i * L, L)] = jnp.exp(ee)
                return 0
            lax.fori_loop(0, SC_E // L, exloop, 0, unroll=4)

            def gixloop(i, _):
                r_ = i // (K // L)
                o_ = (i % (K // L)) * L
                sv = src1d[pl.ds(i * L, L)]
                dv = dst1d[pl.ds(i * L, L)]
                gix[r_, pl.ds(o_, L)] = sv * stride + off
                dix[r_, pl.ds(o_, L)] = dv
                return 0
            lax.fori_loop(0, SC_E // L, gixloop, 0, unroll=4)

            def start_gather(r):
                return pltpu.async_copy(
                    table.at[gix.at[r]], rowbuf.at[r % 2], gsem)

            def scale_chunk(r):
                def scale(e_, _):
                    exs = plsc.load_gather(
                        ex1d, [jnp.full((L,), r * K + e_, jnp.int32)])
                    for j in range(NV):
                        rowbuf[r % 2, e_, pl.ds(j * L, L)] = (
                            rowbuf[r % 2, e_, pl.ds(j * L, L)] * exs)
                    return 0
                lax.fori_loop(0, K, scale, 0, unroll=4)

            pend_s = [None] * NCK
            pend_d = [None] * NCK
            g_cur = start_gather(0)
            for r in range(NCK):
                if r + 1 < NCK:
                    if r >= 1:
                        pend_s[r - 1].wait()
                        pend_d[r - 1].wait()
                    g_next = start_gather(r + 1)
                g_cur.wait()
                scale_chunk(r)
                pend_s[r] = pltpu.async_copy(
                    rowbuf.at[r % 2], acc_sh.at[dix.at[r]], ssem, add=True)
                pend_d[r] = pltpu.async_copy(
                    ex1d.at[pl.ds(r * K, K)], den_sh.at[dix.at[r]], dsem,
                    add=True)
                if r + 1 < NCK:
                    g_cur = g_next
            pend_s[NCK - 2].wait()
            pend_d[NCK - 2].wait()
            pend_s[NCK - 1].wait()
            pend_d[NCK - 1].wait()
            return 0

        lax.fori_loop(0, NSC, superchunk, 0)
        plsc.subcore_barrier()

        pltpu.sync_copy(den_sh.at[pl.ds(rbase, RT)], denbuf)
        col0 = pl.multiple_of(h * jnp.int32(F) + c * FH, FH)
        for r in range(NRC):
            pltpu.sync_copy(acc_sh.at[pl.ds(rbase + r * RC, RC)], cbuf)

            def scale_out(i, _):
                den = plsc.load_gather(
                    denbuf, [jnp.full((L,), r * RC + i, jnp.int32)])
                rec = jnp.float32(1.0) / (den + jnp.float32(1e-16))
                for j in range(NV):
                    cbuf[i, pl.ds(j * L, L)] = cbuf[i, pl.ds(j * L, L)] * rec
                return 0
            lax.fori_loop(0, RC, scale_out, 0, unroll=4)
            pltpu.sync_copy(
                cbuf,
                out_hbm.at[pl.ds(rbase + r * RC, RC), pl.ds(col0, FH)])
        lax.fori_loop(0, RC, zero_cbuf, 0)
        lax.fori_loop(0, RT // L, zero_den, 0)
        return 0

    lax.fori_loop(0, nh, head, 0)


def _make_sc_gal(nh):
    mesh = plsc.VectorSubcoreMesh(core_axis_name="c", subcore_axis_name="s",
                                  num_cores=NC, num_subcores=NS)
    scratch = [
        pltpu.VMEM((SC_E,), jnp.int32),
        pltpu.VMEM((SC_E,), jnp.int32),
        pltpu.VMEM((NCK, K), jnp.int32),
        pltpu.VMEM((NCK, K), jnp.int32),
        pltpu.VMEM((SC_E,), jnp.float32),
        pltpu.VMEM((SC_E,), jnp.float32),
        pltpu.VMEM((SC_E,), jnp.float32),
        pltpu.VMEM((2, K, FH), jnp.float32),
        pltpu.VMEM((RC, FH), jnp.float32),
        pltpu.VMEM((RT,), jnp.float32),
        pltpu.VMEM_SHARED((NP,), jnp.float32),
        pltpu.VMEM_SHARED((NP,), jnp.float32),
        pltpu.VMEM_SHARED((NP, FH), jnp.float32),
        pltpu.VMEM_SHARED((NP,), jnp.float32),
        pltpu.SemaphoreType.DMA,
        pltpu.SemaphoreType.DMA,
        pltpu.SemaphoreType.DMA,
        pltpu.SemaphoreType.DMA,
    ]
    body = functools.partial(_sc_gal_body, nh)

    def run(table, src, dst, s1, s2):
        return pl.kernel(
            body,
            out_type=jax.ShapeDtypeStruct((NP, F * nh), jnp.float32),
            mesh=mesh,
            compiler_params=pltpu.CompilerParams(needs_layout_passes=False),
            scratch_types=scratch,
        )(table, src, dst, s1, s2)

    return run


def _tc_scorevec_body(w0, w1, w2, w3, a0, a1, a2, a3, wo, ao, v8, vo):
    ws = (w0, w1, w2, w3)
    as_ = (a0, a1, a2, a3)
    for h in range(NH):
        v8[:, 2 * h:2 * h + 2] = jnp.dot(
            ws[h][...], as_[h][...], preferred_element_type=jnp.float32)
    vo[...] = jnp.dot(wo[...], ao[...], preferred_element_type=jnp.float32)


def _tc_feat_body(x_ref, wcat_ref, v8_ref, h_ref, s_ref):
    xb = x_ref[...]
    h_ref[...] = jnp.dot(xb, wcat_ref[...], preferred_element_type=jnp.float32)
    s_ref[...] = jnp.dot(xb, v8_ref[...], preferred_element_type=jnp.float32)


def _tc_out_body(hcat_ref, wout_ref, vo_ref, h_ref, s_ref):
    hb = hcat_ref[...]
    h_ref[...] = jnp.dot(hb, wout_ref[...], preferred_element_type=jnp.float32)
    s_ref[...] = jnp.dot(hb, vo_ref[...], preferred_element_type=jnp.float32)


def _tc_act_body(g_ref, o_ref):
    g = g_ref[...]
    g = jnp.where(g > 0.0, g, jnp.exp(g) - 1.0)
    m = jnp.max(g, axis=1, keepdims=True)
    ex = jnp.exp(g - m)
    o_ref[...] = ex / jnp.sum(ex, axis=1, keepdims=True)


def kernel(x, edge_index, dropout, W0, a0, W1, a1, W2, a2, W3, a3,
           W_out, a_out):
    f32 = jnp.float32
    src = edge_index[0]
    dst = edge_index[1]

    A2 = [jnp.stack([a[:F], a[F:]], axis=1) for a in (a0, a1, a2, a3)]
    Ao2 = jnp.stack([a_out[:F], a_out[F:]], axis=1)
    Wcat = jnp.concatenate([W0, W1, W2, W3], axis=1)

    v8, vo = pl.pallas_call(
        _tc_scorevec_body,
        out_shape=(jax.ShapeDtypeStruct((F, 2 * NH), f32),
                   jax.ShapeDtypeStruct((NH * F, 2), f32)),
    )(W0, W1, W2, W3, A2[0], A2[1], A2[2], A2[3], W_out, Ao2)

    xp = jnp.pad(x, ((0, NP - N), (0, 0)))
    R = 1024
    grid = (NP // R,)
    H, S = pl.pallas_call(
        _tc_feat_body,
        grid=grid,
        in_specs=[
            pl.BlockSpec((R, F), lambda i: (i, 0)),
            pl.BlockSpec((F, NH * F), lambda i: (0, 0)),
            pl.BlockSpec((F, 2 * NH), lambda i: (0, 0)),
        ],
        out_specs=[
            pl.BlockSpec((R, NH * F), lambda i: (i, 0)),
            pl.BlockSpec((R, 2 * NH), lambda i: (i, 0)),
        ],
        out_shape=(jax.ShapeDtypeStruct((NP, NH * F), f32),
                   jax.ShapeDtypeStruct((NP, 2 * NH), f32)),
    )(xp, Wcat, v8)

    table4 = H.reshape(NP * 2 * NH, FH)
    s1h = jnp.transpose(S[:, 0::2]).reshape(-1)
    s2h = jnp.transpose(S[:, 1::2]).reshape(-1)
    hcat = _sc_gal4(table4, src, dst, s1h, s2h)

    ho, So = pl.pallas_call(
        _tc_out_body,
        grid=grid,
        in_specs=[
            pl.BlockSpec((R, NH * F), lambda i: (i, 0)),
            pl.BlockSpec((NH * F, F), lambda i: (0, 0)),
            pl.BlockSpec((NH * F, 2), lambda i: (0, 0)),
        ],
        out_specs=[
            pl.BlockSpec((R, F), lambda i: (i, 0)),
            pl.BlockSpec((R, 2), lambda i: (i, 0)),
        ],
        out_shape=(jax.ShapeDtypeStruct((NP, F), f32),
                   jax.ShapeDtypeStruct((NP, 2), f32)),
    )(hcat, W_out, vo)

    table1 = ho.reshape(NP * 2, FH)
    s1o = So[:, 0]
    s2o = So[:, 1]
    g = _sc_gal1(table1, src, dst, s1o, s2o)

    out = pl.pallas_call(
        _tc_act_body,
        grid=grid,
        in_specs=[pl.BlockSpec((R, F), lambda i: (i, 0))],
        out_specs=pl.BlockSpec((R, F), lambda i: (i, 0)),
        out_shape=jax.ShapeDtypeStruct((NP, F), f32),
    )(g)
    return out[:N]


_sc_gal4 = _make_sc_gal(NH)
_sc_gal1 = _make_sc_gal(1)


# speedup vs baseline: 17.4993x; 1.1059x over previous
import functools

import jax
import jax.numpy as jnp
from jax import lax
from jax.experimental import pallas as pl
from jax.experimental.pallas import tpu as pltpu
from jax.experimental.pallas import tpu_sc as plsc

N = 10000
NP = 10240
E = 160000
F = 256
NH = 4
L = 16
NC = 2
NS = 16
ET = E // NS
K = 80
NCHUNK = ET // K
RT = NP // NS
RC = 40
NRC = RT // RC
FH = F // NC
NV = FH // L


SC_E = 2000
NSC = ET // SC_E
NCK = SC_E // K


def _sc_gal_body(nh, table, src_hbm, dst_hbm, s1_hbm, s2_hbm, out_hbm,
                 src1d, dst1d, gix, dix, s1g, s2g, ex1d, rowbuf, cbuf, denbuf,
                 s1_sh, s2_sh, acc_sh, den_sh, gsem, ssem, dsem, xsem):
    c = lax.axis_index("c")
    s = lax.axis_index("s")
    rbase = pl.multiple_of(s * RT, 8)

    zeros16 = jnp.zeros((L,), jnp.float32)

    def zero_cbuf(i, _):
        for j in range(NV):
            cbuf[i, pl.ds(j * L, L)] = zeros16
        return 0

    def zero_den(i, _):
        denbuf[pl.ds(i * L, L)] = zeros16
        return 0

    lax.fori_loop(0, RC, zero_cbuf, 0)
    lax.fori_loop(0, RT // L, zero_den, 0)

    def head(h, _):
        for r in range(NRC):
            pltpu.sync_copy(cbuf, acc_sh.at[pl.ds(rbase + r * RC, RC)])
        pltpu.sync_copy(denbuf, den_sh.at[pl.ds(rbase, RT)])

        sbase = pl.multiple_of(h * NP, 8)

        @pl.when(s == 0)
        def _load_scores():
            pltpu.sync_copy(s1_hbm.at[pl.ds(sbase, NP)], s1_sh)
            pltpu.sync_copy(s2_hbm.at[pl.ds(sbase, NP)], s2_sh)

        plsc.subcore_barrier()

        stride = jnp.int32(2 * nh)
        off = jnp.int32(2) * h + c

        def superchunk(u, _):
            eb = pl.multiple_of(s * ET + u * SC_E, 8)
            pltpu.sync_copy(src_hbm.at[pl.ds(eb, SC_E)], src1d)
            pltpu.sync_copy(dst_hbm.at[pl.ds(eb, SC_E)], dst1d)
            pltpu.async_copy(s1_sh.at[src1d], s1g, xsem).wait()
            pltpu.async_copy(s2_sh.at[dst1d], s2g, xsem).wait()

            def exloop(i, _):
                uu = s1g[pl.ds(i * L, L)] + s2g[pl.ds(i * L, L)]
                ee = jnp.where(uu >= 0.0, uu, uu * jnp.float32(0.2))
                ex1d[pl.ds(i * L, L)] = jnp.exp(ee)
                return 0
            lax.fori_loop(0, SC_E // L, exloop, 0, unroll=4)

            def gixloop(i, _):
                r_ = i // (K // L)
                o_ = (i % (K // L)) * L
                sv = src1d[pl.ds(i * L, L)]
                dv = dst1d[pl.ds(i * L, L)]
                gix[r_, pl.ds(o_, L)] = sv * stride + off
                dix[r_, pl.ds(o_, L)] = dv
                return 0
            lax.fori_loop(0, SC_E // L, gixloop, 0, unroll=4)

            def start_gather(r):
                return pltpu.async_copy(
                    table.at[gix.at[r]], rowbuf.at[r % 2], gsem)

            def scale_chunk(r):
                def scale_g(g, _):
                    exv = ex1d[pl.ds(r * K + g * L, L)]

                    def scale_e(e_, _2):
                        exs = jnp.take(exv, jnp.full((L,), e_, jnp.int32))
                        ge = g * L + e_
                        for j in range(NV):
                            rowbuf[r % 2, ge, pl.ds(j * L, L)] = (
                                rowbuf[r % 2, ge, pl.ds(j * L, L)] * exs)
                        return 0
                    lax.fori_loop(0, L, scale_e, 0, unroll=2)
                    return 0
                lax.fori_loop(0, K // L, scale_g, 0)

            pend_s = [None] * NCK
            pend_d = [None] * NCK
            g_cur = start_gather(0)
            for r in range(NCK):
                if r + 1 < NCK:
                    if r >= 1:
                        pend_s[r - 1].wait()
                        pend_d[r - 1].wait()
                    g_next = start_gather(r + 1)
                g_cur.wait()
                scale_chunk(r)
                pend_s[r] = pltpu.async_copy(
                    rowbuf.at[r % 2], acc_sh.at[dix.at[r]], ssem, add=True)
                pend_d[r] = pltpu.async_copy(
                    ex1d.at[pl.ds(r * K, K)], den_sh.at[dix.at[r]], dsem,
                    add=True)
                if r + 1 < NCK:
                    g_cur = g_next
            pend_s[NCK - 2].wait()
            pend_d[NCK - 2].wait()
            pend_s[NCK - 1].wait()
            pend_d[NCK - 1].wait()
            return 0

        lax.fori_loop(0, NSC, superchunk, 0)
        plsc.subcore_barrier()

        pltpu.sync_copy(den_sh.at[pl.ds(rbase, RT)], denbuf)
        col0 = pl.multiple_of(h * jnp.int32(F) + c * FH, FH)
        for r in range(NRC):
            pltpu.sync_copy(acc_sh.at[pl.ds(rbase + r * RC, RC)], cbuf)

            def scale_out(i, _):
                den = plsc.load_gather(
                    denbuf, [jnp.full((L,), r * RC + i, jnp.int32)])
                rec = jnp.float32(1.0) / (den + jnp.float32(1e-16))
                for j in range(NV):
                    cbuf[i, pl.ds(j * L, L)] = cbuf[i, pl.ds(j * L, L)] * rec
                return 0
            lax.fori_loop(0, RC, scale_out, 0, unroll=4)
            pltpu.sync_copy(
                cbuf,
                out_hbm.at[pl.ds(rbase + r * RC, RC), pl.ds(col0, FH)])
        lax.fori_loop(0, RC, zero_cbuf, 0)
        lax.fori_loop(0, RT // L, zero_den, 0)
        return 0

    lax.fori_loop(0, nh, head, 0)


def _make_sc_gal(nh):
    mesh = plsc.VectorSubcoreMesh(core_axis_name="c", subcore_axis_name="s",
                                  num_cores=NC, num_subcores=NS)
    scratch = [
        pltpu.VMEM((SC_E,), jnp.int32),
        pltpu.VMEM((SC_E,), jnp.int32),
        pltpu.VMEM((NCK, K), jnp.int32),
        pltpu.VMEM((NCK, K), jnp.int32),
        pltpu.VMEM((SC_E,), jnp.float32),
        pltpu.VMEM((SC_E,), jnp.float32),
        pltpu.VMEM((SC_E,), jnp.float32),
        pltpu.VMEM((2, K, FH), jnp.float32),
        pltpu.VMEM((RC, FH), jnp.float32),
        pltpu.VMEM((RT,), jnp.float32),
        pltpu.VMEM_SHARED((NP,), jnp.float32),
        pltpu.VMEM_SHARED((NP,), jnp.float32),
        pltpu.VMEM_SHARED((NP, FH), jnp.float32),
        pltpu.VMEM_SHARED((NP,), jnp.float32),
        pltpu.SemaphoreType.DMA,
        pltpu.SemaphoreType.DMA,
        pltpu.SemaphoreType.DMA,
        pltpu.SemaphoreType.DMA,
    ]
    body = functools.partial(_sc_gal_body, nh)

    def run(table, src, dst, s1, s2):
        return pl.kernel(
            body,
            out_type=jax.ShapeDtypeStruct((NP, F * nh), jnp.float32),
            mesh=mesh,
            compiler_params=pltpu.CompilerParams(needs_layout_passes=False),
            scratch_types=scratch,
        )(table, src, dst, s1, s2)

    return run


def _tc_scorevec_body(w0, w1, w2, w3, a0, a1, a2, a3, wo, ao, v8, vo):
    ws = (w0, w1, w2, w3)
    as_ = (a0, a1, a2, a3)
    for h in range(NH):
        v8[:, 2 * h:2 * h + 2] = jnp.dot(
            ws[h][...], as_[h][...], preferred_element_type=jnp.float32)
    vo[...] = jnp.dot(wo[...], ao[...], preferred_element_type=jnp.float32)


def _tc_feat_body(x_ref, wcat_ref, v8_ref, h_ref, s_ref):
    xb = x_ref[...]
    h_ref[...] = jnp.dot(xb, wcat_ref[...], preferred_element_type=jnp.float32)
    s_ref[...] = jnp.dot(xb, v8_ref[...], preferred_element_type=jnp.float32)


def _tc_out_body(hcat_ref, wout_ref, vo_ref, h_ref, s_ref):
    hb = hcat_ref[...]
    h_ref[...] = jnp.dot(hb, wout_ref[...], preferred_element_type=jnp.float32)
    s_ref[...] = jnp.dot(hb, vo_ref[...], preferred_element_type=jnp.float32)


def _tc_act_body(g_ref, o_ref):
    g = g_ref[...]
    g = jnp.where(g > 0.0, g, jnp.exp(g) - 1.0)
    m = jnp.max(g, axis=1, keepdims=True)
    ex = jnp.exp(g - m)
    o_ref[...] = ex / jnp.sum(ex, axis=1, keepdims=True)


def kernel(x, edge_index, dropout, W0, a0, W1, a1, W2, a2, W3, a3,
           W_out, a_out):
    f32 = jnp.float32
    src = edge_index[0]
    dst = edge_index[1]

    A2 = [jnp.stack([a[:F], a[F:]], axis=1) for a in (a0, a1, a2, a3)]
    Ao2 = jnp.stack([a_out[:F], a_out[F:]], axis=1)
    Wcat = jnp.concatenate([W0, W1, W2, W3], axis=1)

    v8, vo = pl.pallas_call(
        _tc_scorevec_body,
        out_shape=(jax.ShapeDtypeStruct((F, 2 * NH), f32),
                   jax.ShapeDtypeStruct((NH * F, 2), f32)),
    )(W0, W1, W2, W3, A2[0], A2[1], A2[2], A2[3], W_out, Ao2)

    xp = jnp.pad(x, ((0, NP - N), (0, 0)))
    R = 1024
    grid = (NP // R,)
    H, S = pl.pallas_call(
        _tc_feat_body,
        grid=grid,
        in_specs=[
            pl.BlockSpec((R, F), lambda i: (i, 0)),
            pl.BlockSpec((F, NH * F), lambda i: (0, 0)),
            pl.BlockSpec((F, 2 * NH), lambda i: (0, 0)),
        ],
        out_specs=[
            pl.BlockSpec((R, NH * F), lambda i: (i, 0)),
            pl.BlockSpec((R, 2 * NH), lambda i: (i, 0)),
        ],
        out_shape=(jax.ShapeDtypeStruct((NP, NH * F), f32),
                   jax.ShapeDtypeStruct((NP, 2 * NH), f32)),
    )(xp, Wcat, v8)

    table4 = H.reshape(NP * 2 * NH, FH)
    s1h = jnp.transpose(S[:, 0::2]).reshape(-1)
    s2h = jnp.transpose(S[:, 1::2]).reshape(-1)
    hcat = _sc_gal4(table4, src, dst, s1h, s2h)

    ho, So = pl.pallas_call(
        _tc_out_body,
        grid=grid,
        in_specs=[
            pl.BlockSpec((R, NH * F), lambda i: (i, 0)),
            pl.BlockSpec((NH * F, F), lambda i: (0, 0)),
            pl.BlockSpec((NH * F, 2), lambda i: (0, 0)),
        ],
        out_specs=[
            pl.BlockSpec((R, F), lambda i: (i, 0)),
            pl.BlockSpec((R, 2), lambda i: (i, 0)),
        ],
        out_shape=(jax.ShapeDtypeStruct((NP, F), f32),
                   jax.ShapeDtypeStruct((NP, 2), f32)),
    )(hcat, W_out, vo)

    table1 = ho.reshape(NP * 2, FH)
    s1o = So[:, 0]
    s2o = So[:, 1]
    g = _sc_gal1(table1, src, dst, s1o, s2o)

    out = pl.pallas_call(
        _tc_act_body,
        grid=grid,
        in_specs=[pl.BlockSpec((R, F), lambda i: (i, 0))],
        out_specs=pl.BlockSpec((R, F), lambda i: (i, 0)),
        out_shape=jax.ShapeDtypeStruct((NP, F), f32),
    )(g)
    return out[:N]


_sc_gal4 = _make_sc_gal(NH)
_sc_gal1 = _make_sc_gal(1)


# speedup vs baseline: 18.5973x; 1.0628x over previous
import functools

import jax
import jax.numpy as jnp
from jax import lax
from jax.experimental import pallas as pl
from jax.experimental.pallas import tpu as pltpu
from jax.experimental.pallas import tpu_sc as plsc

N = 10000
NP = 10240
E = 160000
F = 256
NH = 4
L = 16
NC = 2
NS = 16
ET = E // NS
K = 80
NCHUNK = ET // K
RT = NP // NS
RC = 80
NRC = RT // RC
FH = F // NC
NV = FH // L


SC_E = 2000
NSC = ET // SC_E
NCK = SC_E // K


def _sc_gal_body(nh, table, src_hbm, dst_hbm, s1_hbm, s2_hbm, out_hbm,
                 src1d, dst1d, gix, dix, s1g, s2g, ex1d, rowbuf, denbuf,
                 s1_sh, s2_sh, acc_sh, den_sh, gsem, ssem, dsem, xsem, wsem):
    c = lax.axis_index("c")
    s = lax.axis_index("s")
    rbase = pl.multiple_of(s * RT, 8)

    zeros16 = jnp.zeros((L,), jnp.float32)

    def zero_cbuf(i, _):
        for j in range(NV):
            rowbuf[0, i, pl.ds(j * L, L)] = zeros16
        return 0

    def zero_den(i, _):
        denbuf[pl.ds(i * L, L)] = zeros16
        return 0

    def head(h, _):
        lax.fori_loop(0, RC, zero_cbuf, 0)
        lax.fori_loop(0, RT // L, zero_den, 0)
        zpend = [pltpu.async_copy(rowbuf.at[0],
                                  acc_sh.at[pl.ds(rbase + r * RC, RC)], wsem)
                 for r in range(NRC)]
        zpend.append(pltpu.async_copy(denbuf, den_sh.at[pl.ds(rbase, RT)],
                                      wsem))
        for d in zpend:
            d.wait()

        sbase = pl.multiple_of(h * NP, 8)

        @pl.when(s == 0)
        def _load_scores():
            pltpu.sync_copy(s1_hbm.at[pl.ds(sbase, NP)], s1_sh)
            pltpu.sync_copy(s2_hbm.at[pl.ds(sbase, NP)], s2_sh)

        plsc.subcore_barrier()

        stride = jnp.int32(2 * nh)
        off = jnp.int32(2) * h + c

        def superchunk(u, _):
            eb = pl.multiple_of(s * ET + u * SC_E, 8)
            d1 = pltpu.async_copy(src_hbm.at[pl.ds(eb, SC_E)], src1d, xsem)
            d2 = pltpu.async_copy(dst_hbm.at[pl.ds(eb, SC_E)], dst1d, xsem)
            d1.wait()
            d2.wait()
            d1 = pltpu.async_copy(s1_sh.at[src1d], s1g, xsem)
            d2 = pltpu.async_copy(s2_sh.at[dst1d], s2g, xsem)
            d1.wait()
            d2.wait()

            def exloop(i, _):
                uu = s1g[pl.ds(i * L, L)] + s2g[pl.ds(i * L, L)]
                ee = jnp.where(uu >= 0.0, uu, uu * jnp.float32(0.2))
                ex1d[pl.ds(i * L, L)] = jnp.exp(ee)
                return 0
            lax.fori_loop(0, SC_E // L, exloop, 0, unroll=4)

            def gixloop(i, _):
                r_ = i // (K // L)
                o_ = (i % (K // L)) * L
                sv = src1d[pl.ds(i * L, L)]
                dv = dst1d[pl.ds(i * L, L)]
                gix[r_, pl.ds(o_, L)] = sv * stride + off
                dix[r_, pl.ds(o_, L)] = dv
                return 0
            lax.fori_loop(0, SC_E // L, gixloop, 0, unroll=4)

            def start_gather(r):
                return pltpu.async_copy(
                    table.at[gix.at[r]], rowbuf.at[r % 2], gsem)

            def scale_chunk(r):
                def scale_g(g, _):
                    exv = ex1d[pl.ds(r * K + g * L, L)]

                    def scale_e(e_, _2):
                        exs = jnp.take(exv, jnp.full((L,), e_, jnp.int32))
                        ge = g * L + e_
                        for j in range(NV):
                            rowbuf[r % 2, ge, pl.ds(j * L, L)] = (
                                rowbuf[r % 2, ge, pl.ds(j * L, L)] * exs)
                        return 0
                    lax.fori_loop(0, L, scale_e, 0, unroll=2)
                    return 0
                lax.fori_loop(0, K // L, scale_g, 0)

            pend_s = [None] * NCK
            pend_d = [None] * NCK
            g_cur = start_gather(0)
            for r in range(NCK):
                if r + 1 < NCK:
                    if r >= 1:
                        pend_s[r - 1].wait()
                        pend_d[r - 1].wait()
                    g_next = start_gather(r + 1)
                g_cur.wait()
                scale_chunk(r)
                pend_s[r] = pltpu.async_copy(
                    rowbuf.at[r % 2], acc_sh.at[dix.at[r]], ssem, add=True)
                pend_d[r] = pltpu.async_copy(
                    ex1d.at[pl.ds(r * K, K)], den_sh.at[dix.at[r]], dsem,
                    add=True)
                if r + 1 < NCK:
                    g_cur = g_next
            pend_s[NCK - 2].wait()
            pend_d[NCK - 2].wait()
            pend_s[NCK - 1].wait()
            pend_d[NCK - 1].wait()
            return 0

        lax.fori_loop(0, NSC, superchunk, 0)
        plsc.subcore_barrier()

        pltpu.sync_copy(den_sh.at[pl.ds(rbase, RT)], denbuf)
        col0 = pl.multiple_of(h * jnp.int32(F) + c * FH, FH)

        def rd_cp(r):
            return pltpu.async_copy(
                acc_sh.at[pl.ds(rbase + r * RC, RC)], rowbuf.at[r % 2], gsem)

        def wr_cp(r):
            return pltpu.async_copy(
                rowbuf.at[r % 2],
                out_hbm.at[pl.ds(rbase + r * RC, RC), pl.ds(col0, FH)], wsem)

        wr = [None] * NRC
        rd_cur = rd_cp(0)
        for r in range(NRC):
            if r + 1 < NRC:
                if r >= 1:
                    wr[r - 1].wait()
                rd_nxt = rd_cp(r + 1)
            rd_cur.wait()

            def scale_out(g, _):
                denv = denbuf[pl.ds(r * RC + g * L, L)]
                rec = jnp.float32(1.0) / (denv + jnp.float32(1e-16))

                def scale_row(e_, _2):
                    rc_ = jnp.take(rec, jnp.full((L,), e_, jnp.int32))
                    ge = g * L + e_
                    for j in range(NV):
                        rowbuf[r % 2, ge, pl.ds(j * L, L)] = (
                            rowbuf[r % 2, ge, pl.ds(j * L, L)] * rc_)
                    return 0
                lax.fori_loop(0, L, scale_row, 0, unroll=2)
                return 0
            lax.fori_loop(0, RC // L, scale_out, 0)
            wr[r] = wr_cp(r)
            if r + 1 < NRC:
                rd_cur = rd_nxt
        wr[NRC - 2].wait()
        wr[NRC - 1].wait()
        return 0

    lax.fori_loop(0, nh, head, 0)


def _make_sc_gal(nh):
    mesh = plsc.VectorSubcoreMesh(core_axis_name="c", subcore_axis_name="s",
                                  num_cores=NC, num_subcores=NS)
    scratch = [
        pltpu.VMEM((SC_E,), jnp.int32),
        pltpu.VMEM((SC_E,), jnp.int32),
        pltpu.VMEM((NCK, K), jnp.int32),
        pltpu.VMEM((NCK, K), jnp.int32),
        pltpu.VMEM((SC_E,), jnp.float32),
        pltpu.VMEM((SC_E,), jnp.float32),
        pltpu.VMEM((SC_E,), jnp.float32),
        pltpu.VMEM((2, K, FH), jnp.float32),
        pltpu.VMEM((RT,), jnp.float32),
        pltpu.VMEM_SHARED((NP,), jnp.float32),
        pltpu.VMEM_SHARED((NP,), jnp.float32),
        pltpu.VMEM_SHARED((NP, FH), jnp.float32),
        pltpu.VMEM_SHARED((NP,), jnp.float32),
        pltpu.SemaphoreType.DMA,
        pltpu.SemaphoreType.DMA,
        pltpu.SemaphoreType.DMA,
        pltpu.SemaphoreType.DMA,
        pltpu.SemaphoreType.DMA,
    ]
    body = functools.partial(_sc_gal_body, nh)

    def run(table, src, dst, s1, s2):
        return pl.kernel(
            body,
            out_type=jax.ShapeDtypeStruct((NP, F * nh), jnp.float32),
            mesh=mesh,
            compiler_params=pltpu.CompilerParams(needs_layout_passes=False),
            scratch_types=scratch,
        )(table, src, dst, s1, s2)

    return run


def _tc_scorevec_body(w0, w1, w2, w3, a0, a1, a2, a3, wo, ao, v8, vo):
    ws = (w0, w1, w2, w3)
    as_ = (a0, a1, a2, a3)
    for h in range(NH):
        v8[:, 2 * h:2 * h + 2] = jnp.dot(
            ws[h][...], as_[h][...], preferred_element_type=jnp.float32)
    vo[...] = jnp.dot(wo[...], ao[...], preferred_element_type=jnp.float32)


def _tc_feat_body(x_ref, wcat_ref, v8_ref, h_ref, s_ref):
    xb = x_ref[...]
    h_ref[...] = jnp.dot(xb, wcat_ref[...], preferred_element_type=jnp.float32)
    s_ref[...] = jnp.dot(xb, v8_ref[...], preferred_element_type=jnp.float32)


def _tc_out_body(hcat_ref, wout_ref, vo_ref, h_ref, s_ref):
    hb = hcat_ref[...]
    h_ref[...] = jnp.dot(hb, wout_ref[...], preferred_element_type=jnp.float32)
    s_ref[...] = jnp.dot(hb, vo_ref[...], preferred_element_type=jnp.float32)


def _tc_act_body(g_ref, o_ref):
    g = g_ref[...]
    g = jnp.where(g > 0.0, g, jnp.exp(g) - 1.0)
    m = jnp.max(g, axis=1, keepdims=True)
    ex = jnp.exp(g - m)
    o_ref[...] = ex / jnp.sum(ex, axis=1, keepdims=True)


def kernel(x, edge_index, dropout, W0, a0, W1, a1, W2, a2, W3, a3,
           W_out, a_out):
    f32 = jnp.float32
    src = edge_index[0]
    dst = edge_index[1]

    A2 = [jnp.stack([a[:F], a[F:]], axis=1) for a in (a0, a1, a2, a3)]
    Ao2 = jnp.stack([a_out[:F], a_out[F:]], axis=1)
    Wcat = jnp.concatenate([W0, W1, W2, W3], axis=1)

    v8, vo = pl.pallas_call(
        _tc_scorevec_body,
        out_shape=(jax.ShapeDtypeStruct((F, 2 * NH), f32),
                   jax.ShapeDtypeStruct((NH * F, 2), f32)),
    )(W0, W1, W2, W3, A2[0], A2[1], A2[2], A2[3], W_out, Ao2)

    xp = jnp.pad(x, ((0, NP - N), (0, 0)))
    R = 1024
    grid = (NP // R,)
    H, S = pl.pallas_call(
        _tc_feat_body,
        grid=grid,
        in_specs=[
            pl.BlockSpec((R, F), lambda i: (i, 0)),
            pl.BlockSpec((F, NH * F), lambda i: (0, 0)),
            pl.BlockSpec((F, 2 * NH), lambda i: (0, 0)),
        ],
        out_specs=[
            pl.BlockSpec((R, NH * F), lambda i: (i, 0)),
            pl.BlockSpec((R, 2 * NH), lambda i: (i, 0)),
        ],
        out_shape=(jax.ShapeDtypeStruct((NP, NH * F), f32),
                   jax.ShapeDtypeStruct((NP, 2 * NH), f32)),
    )(xp, Wcat, v8)

    table4 = H.reshape(NP * 2 * NH, FH)
    s1h = jnp.transpose(S[:, 0::2]).reshape(-1)
    s2h = jnp.transpose(S[:, 1::2]).reshape(-1)
    hcat = _sc_gal4(table4, src, dst, s1h, s2h)

    ho, So = pl.pallas_call(
        _tc_out_body,
        grid=grid,
        in_specs=[
            pl.BlockSpec((R, NH * F), lambda i: (i, 0)),
            pl.BlockSpec((NH * F, F), lambda i: (0, 0)),
            pl.BlockSpec((NH * F, 2), lambda i: (0, 0)),
        ],
        out_specs=[
            pl.BlockSpec((R, F), lambda i: (i, 0)),
            pl.BlockSpec((R, 2), lambda i: (i, 0)),
        ],
        out_shape=(jax.ShapeDtypeStruct((NP, F), f32),
                   jax.ShapeDtypeStruct((NP, 2), f32)),
    )(hcat, W_out, vo)

    table1 = ho.reshape(NP * 2, FH)
    s1o = So[:, 0]
    s2o = So[:, 1]
    g = _sc_gal1(table1, src, dst, s1o, s2o)

    out = pl.pallas_call(
        _tc_act_body,
        grid=grid,
        in_specs=[pl.BlockSpec((R, F), lambda i: (i, 0))],
        out_specs=pl.BlockSpec((R, F), lambda i: (i, 0)),
        out_shape=jax.ShapeDtypeStruct((NP, F), f32),
    )(g)
    return out[:N]


_sc_gal4 = _make_sc_gal(NH)
_sc_gal1 = _make_sc_gal(1)


# speedup vs baseline: 20.7967x; 1.1183x over previous
import functools

import jax
import jax.numpy as jnp
from jax import lax
from jax.experimental import pallas as pl
from jax.experimental.pallas import tpu as pltpu
from jax.experimental.pallas import tpu_sc as plsc

N = 10000
NP = 10240
E = 160000
F = 256
NH = 4
L = 16
NC = 2
NS = 16
ET = E // NS
K = 80
NCHUNK = ET // K
RT = NP // NS
RC = 80
NRC = RT // RC
FH = F // NC
NV = FH // L


SC_E = 2000
NSC = ET // SC_E
NCK = SC_E // K


def _sc_gal_body(nh, table, src_hbm, dst_hbm, s1_hbm, s2_hbm, out_hbm,
                 src1d, dst1d, dix, s1g, s2g, ex1d, rowbuf, denbuf,
                 s1_sh, s2_sh, acc_sh, den_sh, gsem, ssem, dsem, xsem, wsem):
    c = lax.axis_index("c")
    s = lax.axis_index("s")
    rbase = pl.multiple_of(s * RT, 8)

    zeros16 = jnp.zeros((L,), jnp.float32)

    def zero_cbuf(i, _):
        for j in range(NV):
            rowbuf[0, i, pl.ds(j * L, L)] = zeros16
        return 0

    def zero_den(i, _):
        denbuf[pl.ds(i * L, L)] = zeros16
        return 0

    def head(h, _):
        lax.fori_loop(0, RC, zero_cbuf, 0)
        lax.fori_loop(0, RT // L, zero_den, 0)
        zpend = [pltpu.async_copy(rowbuf.at[0],
                                  acc_sh.at[pl.ds(rbase + r * RC, RC)], wsem)
                 for r in range(NRC)]
        zpend.append(pltpu.async_copy(denbuf, den_sh.at[pl.ds(rbase, RT)],
                                      wsem))
        for d in zpend:
            d.wait()

        sbase = pl.multiple_of(h * NP, 8)

        @pl.when(s == 0)
        def _load_scores():
            pltpu.sync_copy(s1_hbm.at[pl.ds(sbase, NP)], s1_sh)
            pltpu.sync_copy(s2_hbm.at[pl.ds(sbase, NP)], s2_sh)

        plsc.subcore_barrier()

        stride = jnp.int32(2 * nh)
        off = jnp.int32(2) * h + c

        def superchunk(u, _):
            eb = pl.multiple_of(s * ET + u * SC_E, 8)
            d1 = pltpu.async_copy(src_hbm.at[pl.ds(eb, SC_E)], src1d, xsem)
            d2 = pltpu.async_copy(dst_hbm.at[pl.ds(eb, SC_E)], dst1d, xsem)
            d1.wait()
            d2.wait()
            d1 = pltpu.async_copy(s1_sh.at[src1d], s1g, xsem)
            d2 = pltpu.async_copy(s2_sh.at[dst1d], s2g, xsem)
            d1.wait()
            d2.wait()

            def exloop(i, _):
                uu = s1g[pl.ds(i * L, L)] + s2g[pl.ds(i * L, L)]
                ee = jnp.where(uu >= 0.0, uu, uu * jnp.float32(0.2))
                ex1d[pl.ds(i * L, L)] = jnp.exp(ee)
                return 0
            lax.fori_loop(0, SC_E // L, exloop, 0, unroll=4)

            def gixloop(i, _):
                r_ = i // (K // L)
                o_ = (i % (K // L)) * L
                sv = src1d[pl.ds(i * L, L)]
                dv = dst1d[pl.ds(i * L, L)]
                src1d[pl.ds(i * L, L)] = sv * stride + off
                dix[r_, pl.ds(o_, L)] = dv
                return 0
            lax.fori_loop(0, SC_E // L, gixloop, 0, unroll=4)

            def start_gather(r):
                return pltpu.async_copy(
                    table.at[src1d.at[pl.ds(r * K, K)]], rowbuf.at[r % 3],
                    gsem)

            def scale_chunk(r):
                def scale_g(g, _):
                    exv = ex1d[pl.ds(r * K + g * L, L)]

                    def scale_e(e_, _2):
                        exs = jnp.take(exv, jnp.full((L,), e_, jnp.int32))
                        ge = g * L + e_
                        for j in range(NV):
                            rowbuf[r % 3, ge, pl.ds(j * L, L)] = (
                                rowbuf[r % 3, ge, pl.ds(j * L, L)] * exs)
                        return 0
                    lax.fori_loop(0, L, scale_e, 0, unroll=2)
                    return 0
                lax.fori_loop(0, K // L, scale_g, 0)

            pend_s = [None] * NCK
            pend_d = [None] * NCK
            g_cur = start_gather(0)
            for r in range(NCK):
                if r + 1 < NCK:
                    if r >= 2:
                        pend_s[r - 2].wait()
                        pend_d[r - 2].wait()
                    g_next = start_gather(r + 1)
                g_cur.wait()
                scale_chunk(r)
                pend_s[r] = pltpu.async_copy(
                    rowbuf.at[r % 3], acc_sh.at[dix.at[r]], ssem, add=True)
                pend_d[r] = pltpu.async_copy(
                    ex1d.at[pl.ds(r * K, K)], den_sh.at[dix.at[r]], dsem,
                    add=True)
                if r + 1 < NCK:
                    g_cur = g_next
            for rr in range(NCK - 3, NCK):
                pend_s[rr].wait()
                pend_d[rr].wait()
            return 0

        lax.fori_loop(0, NSC, superchunk, 0)
        plsc.subcore_barrier()

        pltpu.sync_copy(den_sh.at[pl.ds(rbase, RT)], denbuf)
        col0 = pl.multiple_of(h * jnp.int32(F) + c * FH, FH)

        def rd_cp(r):
            return pltpu.async_copy(
                acc_sh.at[pl.ds(rbase + r * RC, RC)], rowbuf.at[r % 3], gsem)

        def wr_cp(r):
            return pltpu.async_copy(
                rowbuf.at[r % 3],
                out_hbm.at[pl.ds(rbase + r * RC, RC), pl.ds(col0, FH)], wsem)

        wr = [None] * NRC
        rd_cur = rd_cp(0)
        for r in range(NRC):
            if r + 1 < NRC:
                if r >= 2:
                    wr[r - 2].wait()
                rd_nxt = rd_cp(r + 1)
            rd_cur.wait()

            def scale_out(g, _):
                denv = denbuf[pl.ds(r * RC + g * L, L)]
                rec = jnp.float32(1.0) / (denv + jnp.float32(1e-16))

                def scale_row(e_, _2):
                    rc_ = jnp.take(rec, jnp.full((L,), e_, jnp.int32))
                    ge = g * L + e_
                    for j in range(NV):
                        rowbuf[r % 3, ge, pl.ds(j * L, L)] = (
                            rowbuf[r % 3, ge, pl.ds(j * L, L)] * rc_)
                    return 0
                lax.fori_loop(0, L, scale_row, 0, unroll=2)
                return 0
            lax.fori_loop(0, RC // L, scale_out, 0)
            wr[r] = wr_cp(r)
            if r + 1 < NRC:
                rd_cur = rd_nxt
        for rr in range(NRC - 3, NRC):
            wr[rr].wait()
        return 0

    lax.fori_loop(0, nh, head, 0)


def _make_sc_gal(nh):
    mesh = plsc.VectorSubcoreMesh(core_axis_name="c", subcore_axis_name="s",
                                  num_cores=NC, num_subcores=NS)
    scratch = [
        pltpu.VMEM((SC_E,), jnp.int32),
        pltpu.VMEM((SC_E,), jnp.int32),
        pltpu.VMEM((NCK, K), jnp.int32),
        pltpu.VMEM((SC_E,), jnp.float32),
        pltpu.VMEM((SC_E,), jnp.float32),
        pltpu.VMEM((SC_E,), jnp.float32),
        pltpu.VMEM((3, K, FH), jnp.float32),
        pltpu.VMEM((RT,), jnp.float32),
        pltpu.VMEM_SHARED((NP,), jnp.float32),
        pltpu.VMEM_SHARED((NP,), jnp.float32),
        pltpu.VMEM_SHARED((NP, FH), jnp.float32),
        pltpu.VMEM_SHARED((NP,), jnp.float32),
        pltpu.SemaphoreType.DMA,
        pltpu.SemaphoreType.DMA,
        pltpu.SemaphoreType.DMA,
        pltpu.SemaphoreType.DMA,
        pltpu.SemaphoreType.DMA,
    ]
    body = functools.partial(_sc_gal_body, nh)

    def run(table, src, dst, s1, s2):
        return pl.kernel(
            body,
            out_type=jax.ShapeDtypeStruct((NP, F * nh), jnp.float32),
            mesh=mesh,
            compiler_params=pltpu.CompilerParams(needs_layout_passes=False),
            scratch_types=scratch,
        )(table, src, dst, s1, s2)

    return run


def _tc_scorevec_body(w0, w1, w2, w3, a0, a1, a2, a3, wo, ao, v8, vo):
    ws = (w0, w1, w2, w3)
    as_ = (a0, a1, a2, a3)
    for h in range(NH):
        v8[:, 2 * h:2 * h + 2] = jnp.dot(
            ws[h][...], as_[h][...], preferred_element_type=jnp.float32)
    vo[...] = jnp.dot(wo[...], ao[...], preferred_element_type=jnp.float32)


def _tc_feat_body(x_ref, wcat_ref, v8_ref, h_ref, s_ref):
    xb = x_ref[...]
    h_ref[...] = jnp.dot(xb, wcat_ref[...], preferred_element_type=jnp.float32)
    s_ref[...] = jnp.dot(xb, v8_ref[...], preferred_element_type=jnp.float32)


def _tc_out_body(hcat_ref, wout_ref, vo_ref, h_ref, s_ref):
    hb = hcat_ref[...]
    h_ref[...] = jnp.dot(hb, wout_ref[...], preferred_element_type=jnp.float32)
    s_ref[...] = jnp.dot(hb, vo_ref[...], preferred_element_type=jnp.float32)


def _tc_act_body(g_ref, o_ref):
    g = g_ref[...]
    g = jnp.where(g > 0.0, g, jnp.exp(g) - 1.0)
    m = jnp.max(g, axis=1, keepdims=True)
    ex = jnp.exp(g - m)
    o_ref[...] = ex / jnp.sum(ex, axis=1, keepdims=True)


def kernel(x, edge_index, dropout, W0, a0, W1, a1, W2, a2, W3, a3,
           W_out, a_out):
    f32 = jnp.float32
    src = edge_index[0]
    dst = edge_index[1]

    A2 = [jnp.stack([a[:F], a[F:]], axis=1) for a in (a0, a1, a2, a3)]
    Ao2 = jnp.stack([a_out[:F], a_out[F:]], axis=1)
    Wcat = jnp.concatenate([W0, W1, W2, W3], axis=1)

    v8, vo = pl.pallas_call(
        _tc_scorevec_body,
        out_shape=(jax.ShapeDtypeStruct((F, 2 * NH), f32),
                   jax.ShapeDtypeStruct((NH * F, 2), f32)),
    )(W0, W1, W2, W3, A2[0], A2[1], A2[2], A2[3], W_out, Ao2)

    xp = jnp.pad(x, ((0, NP - N), (0, 0)))
    R = 1024
    grid = (NP // R,)
    H, S = pl.pallas_call(
        _tc_feat_body,
        grid=grid,
        in_specs=[
            pl.BlockSpec((R, F), lambda i: (i, 0)),
            pl.BlockSpec((F, NH * F), lambda i: (0, 0)),
            pl.BlockSpec((F, 2 * NH), lambda i: (0, 0)),
        ],
        out_specs=[
            pl.BlockSpec((R, NH * F), lambda i: (i, 0)),
            pl.BlockSpec((R, 2 * NH), lambda i: (i, 0)),
        ],
        out_shape=(jax.ShapeDtypeStruct((NP, NH * F), f32),
                   jax.ShapeDtypeStruct((NP, 2 * NH), f32)),
    )(xp, Wcat, v8)

    table4 = H.reshape(NP * 2 * NH, FH)
    s1h = jnp.transpose(S[:, 0::2]).reshape(-1)
    s2h = jnp.transpose(S[:, 1::2]).reshape(-1)
    hcat = _sc_gal4(table4, src, dst, s1h, s2h)

    ho, So = pl.pallas_call(
        _tc_out_body,
        grid=grid,
        in_specs=[
            pl.BlockSpec((R, NH * F), lambda i: (i, 0)),
            pl.BlockSpec((NH * F, F), lambda i: (0, 0)),
            pl.BlockSpec((NH * F, 2), lambda i: (0, 0)),
        ],
        out_specs=[
            pl.BlockSpec((R, F), lambda i: (i, 0)),
            pl.BlockSpec((R, 2), lambda i: (i, 0)),
        ],
        out_shape=(jax.ShapeDtypeStruct((NP, F), f32),
                   jax.ShapeDtypeStruct((NP, 2), f32)),
    )(hcat, W_out, vo)

    table1 = ho.reshape(NP * 2, FH)
    s1o = So[:, 0]
    s2o = So[:, 1]
    g = _sc_gal1(table1, src, dst, s1o, s2o)

    out = pl.pallas_call(
        _tc_act_body,
        grid=grid,
        in_specs=[pl.BlockSpec((R, F), lambda i: (i, 0))],
        out_specs=pl.BlockSpec((R, F), lambda i: (i, 0)),
        out_shape=jax.ShapeDtypeStruct((NP, F), f32),
    )(g)
    return out[:N]


_sc_gal4 = _make_sc_gal(NH)
_sc_gal1 = _make_sc_gal(1)
